# baseline (speedup 1.0000x reference)
import functools

import jax
import jax.numpy as jnp
from jax import lax
from jax.experimental import pallas as pl
from jax.experimental.pallas import tpu as pltpu
from jax.experimental.pallas import tpu_sc as plsc

N_NODES = 10000
N_EDGES = 320000

NC = 2
NS = 16
NW = NC * NS
CH = 128
EPW = 10240
NCHUNK = EPW // CH
E_PAD = NW * EPW
NROWS = 10240
STRIPE = NROWS // NS
TRASH = N_NODES
D = 128


def _make_agg():
  mesh = plsc.VectorSubcoreMesh(
      core_axis_name="c", subcore_axis_name="s", num_cores=NC, num_subcores=NS)

  @functools.partial(
      pl.kernel,
      out_type=jax.ShapeDtypeStruct((NC, NROWS, D), jnp.float32),
      mesh=mesh,
      scratch_types=[
          pltpu.VMEM((NCHUNK, CH), jnp.int32),
          pltpu.VMEM((NCHUNK, CH), jnp.int32),
          pltpu.VMEM((CH, D), jnp.float32),
          pltpu.VMEM_SHARED((NROWS, D), jnp.float32),
          pltpu.SemaphoreType.DMA,
      ],
  )
  def agg(u_hbm, src_hbm, dst_hbm, zero_hbm, out_hbm,
          src_v, dst_v, msg_v, acc, sem):
    c = lax.axis_index("c")
    s = lax.axis_index("s")
    wid = c * NS + s
    base = s * STRIPE

    pltpu.sync_copy(src_hbm.at[wid], src_v)
    pltpu.sync_copy(dst_hbm.at[wid], dst_v)
    pltpu.sync_copy(zero_hbm, msg_v)
    for k in range(STRIPE // CH):
      pltpu.sync_copy(msg_v, acc.at[pl.ds(base + k * CH, CH)])
    plsc.subcore_barrier()

    def body(j, carry):
      pltpu.async_copy(u_hbm.at[src_v.at[j]], msg_v, sem).wait()
      pltpu.sync_copy(msg_v, acc.at[dst_v.at[j]], add=True)
      return carry

    lax.fori_loop(0, NCHUNK, body, 0)
    plsc.subcore_barrier()

    for k in range(STRIPE // CH):
      pltpu.sync_copy(acc.at[pl.ds(base + k * CH, CH)], msg_v)
      pltpu.sync_copy(msg_v, out_hbm.at[c, pl.ds(base + k * CH, CH)])

  return agg


_agg_call = _make_agg()


def _agg(u, src3, dst3, zero):
  return _agg_call(u, src3, dst3, zero)


R = 1000
GRID = (N_NODES // R,)


def _row_spec(d):
  return pl.BlockSpec((R, d), lambda i: (i, 0))


def _full_spec(shape):
  return pl.BlockSpec(shape, lambda i: tuple(0 for _ in shape))


def _tc_pre_body(d0_ref, d1_ref, x_ref, dinv_ref, u1_ref):
  deg = d0_ref[...] + d1_ref[...] + 1.0
  dv = lax.rsqrt(deg)
  dinv_ref[...] = dv
  u1_ref[...] = dv * x_ref[...]


def _tc_pre(d0, d1, x):
  return pl.pallas_call(
      _tc_pre_body,
      grid=GRID,
      in_specs=[_row_spec(1), _row_spec(1), _row_spec(128)],
      out_specs=[_row_spec(1), _row_spec(128)],
      out_shape=[
          jax.ShapeDtypeStruct((N_NODES, 1), jnp.float32),
          jax.ShapeDtypeStruct((N_NODES, 128), jnp.float32),
      ],
  )(d0, d1, x)


def _tc1_body(p0, p1, u, dinv, W1, b1, W2, o):
  dv = dinv[...]
  a = dv * (p0[...] + p1[...] + u[...])
  h = jnp.maximum(jnp.dot(a, W1[...], preferred_element_type=jnp.float32)
                  + b1[...], 0.0)
  o[...] = dv * jnp.dot(h, W2[...], preferred_element_type=jnp.float32)


def _tc1(p0, p1, u, dinv, W1, b1, W2):
  return pl.pallas_call(
      _tc1_body,
      grid=GRID,
      in_specs=[_row_spec(128), _row_spec(128), _row_spec(128), _row_spec(1),
                _full_spec((128, 256)), _full_spec((1, 256)),
                _full_spec((256, 128))],
      out_specs=_row_spec(128),
      out_shape=jax.ShapeDtypeStruct((N_NODES, 128), jnp.float32),
  )(p0, p1, u, dinv, W1, b1, W2)


def _tc_mid_body(p0, p1, u, dinv, b, Wn, o):
  dv = dinv[...]
  h = jnp.maximum(dv * (p0[...] + p1[...] + u[...]) + b[...], 0.0)
  o[...] = dv * jnp.dot(h, Wn[...], preferred_element_type=jnp.float32)


def _tc_mid(p0, p1, u, dinv, b, Wn):
  return pl.pallas_call(
      _tc_mid_body,
      grid=GRID,
      in_specs=[_row_spec(128), _row_spec(128), _row_spec(128), _row_spec(1),
                _full_spec((1, 128)), _full_spec((128, 128))],
      out_specs=_row_spec(128),
      out_shape=jax.ShapeDtypeStruct((N_NODES, 128), jnp.float32),
  )(p0, p1, u, dinv, b, Wn)


def _tc_h4_body(p0, p1, u, dinv, b, o):
  dv = dinv[...]
  h = jnp.maximum(dv * (p0[...] + p1[...] + u[...]) + b[...], 0.0)
  o[...] = dv * h


def _tc_h4(p0, p1, u, dinv, b):
  return pl.pallas_call(
      _tc_h4_body,
      grid=GRID,
      in_specs=[_row_spec(128), _row_spec(128), _row_spec(128), _row_spec(1),
                _full_spec((1, 128))],
      out_specs=_row_spec(128),
      out_shape=jax.ShapeDtypeStruct((N_NODES, 128), jnp.float32),
  )(p0, p1, u, dinv, b)


def _tc_fin_body(p0, p1, u, dinv, Wm, bm, Wl, bl, mu, ls):
  a = dinv[...] * (p0[...] + p1[...] + u[...])
  mu[...] = jnp.dot(a, Wm[...], preferred_element_type=jnp.float32) + bm[...]
  ls[...] = jnp.dot(a, Wl[...], preferred_element_type=jnp.float32) + bl[...]


def _tc_fin(p0, p1, u, dinv, Wm, bm, Wl, bl):
  return pl.pallas_call(
      _tc_fin_body,
      grid=GRID,
      in_specs=[_row_spec(128), _row_spec(128), _row_spec(128), _row_spec(1),
                _full_spec((128, 16)), _full_spec((1, 16)),
                _full_spec((128, 16)), _full_spec((1, 16))],
      out_specs=[_row_spec(16), _row_spec(16)],
      out_shape=[
          jax.ShapeDtypeStruct((N_NODES, 16), jnp.float32),
          jax.ShapeDtypeStruct((N_NODES, 16), jnp.float32),
      ],
  )(p0, p1, u, dinv, Wm, bm, Wl, bl)


def _padc(W, cols):
  return jnp.pad(W, ((0, 0), (0, cols - W.shape[1])))


def _padrc(W, rows, cols):
  return jnp.pad(W, ((0, rows - W.shape[0]), (0, cols - W.shape[1])))


def kernel(x, edge_index, W1, b1, W2, b2, W3, b3, W4, b4,
           W_mu, b_mu, W_logstd, b_logstd):
  src = edge_index[0].astype(jnp.int32)
  dst = edge_index[1].astype(jnp.int32)
  pad = E_PAD - N_EDGES
  src3 = jnp.concatenate([src, jnp.zeros((pad,), jnp.int32)]).reshape(
      NW, NCHUNK, CH)
  dst3 = jnp.concatenate([dst, jnp.full((pad,), TRASH, jnp.int32)]).reshape(
      NW, NCHUNK, CH)
  zero = jnp.zeros((CH, D), jnp.float32)

  b1r = b1.reshape(1, -1)
  b2r = b2.reshape(1, -1)
  b3r = _padc(b3.reshape(1, -1), 128)
  b4r = _padc(b4.reshape(1, -1), 128)
  W3p = _padc(W3, 128)
  W4p = _padrc(W4, 128, 128)
  Wmp = _padrc(W_mu, 128, 16)
  Wlp = _padrc(W_logstd, 128, 16)
  bmr = b_mu.reshape(1, -1)
  blr = b_logstd.reshape(1, -1)

  ones = jnp.ones((N_NODES, D), jnp.float32)
  degp = _agg(ones, src3, dst3, zero)
  d0 = degp[0, :N_NODES, :1]
  d1 = degp[1, :N_NODES, :1]
  dinv, u1 = _tc_pre(d0, d1, x)

  p = _agg(u1, src3, dst3, zero)
  u2 = _tc1(p[0, :N_NODES], p[1, :N_NODES], u1, dinv, W1, b1r, W2)

  p = _agg(u2, src3, dst3, zero)
  u3 = _tc_mid(p[0, :N_NODES], p[1, :N_NODES], u2, dinv, b2r, W3p)

  p = _agg(u3, src3, dst3, zero)
  u4 = _tc_mid(p[0, :N_NODES], p[1, :N_NODES], u3, dinv, b3r, W4p)

  p = _agg(u4, src3, dst3, zero)
  u5 = _tc_h4(p[0, :N_NODES], p[1, :N_NODES], u4, dinv, b4r)

  p = _agg(u5, src3, dst3, zero)
  mu, logstd = _tc_fin(p[0, :N_NODES], p[1, :N_NODES], u5, dinv,
                       Wmp, bmr, Wlp, blr)
  return (mu, logstd)


# speedup vs baseline: 1.8208x; 1.8208x over previous
import functools

import jax
import jax.numpy as jnp
from jax import lax
from jax.experimental import pallas as pl
from jax.experimental.pallas import tpu as pltpu
from jax.experimental.pallas import tpu_sc as plsc

N_NODES = 10000
N_EDGES = 320000

NC = 2
NS = 16
NW = NC * NS
NROWS = 10240
STRIPE = NROWS // NS
TRASH = N_NODES

_PLAN = {
    128: dict(ch=64, n0=242, n1=78),
    64: dict(ch=128, n0=120, n1=40),
    32: dict(ch=128, n0=112, n1=48),
    16: dict(ch=128, n0=96, n1=64),
}


def _make_agg(D):
  plan = _PLAN[D]
  ch, n0, n1 = plan["ch"], plan["n0"], plan["n1"]
  mesh = plsc.VectorSubcoreMesh(
      core_axis_name="c", subcore_axis_name="s", num_cores=NC, num_subcores=NS)

  @functools.partial(
      pl.kernel,
      out_type=jax.ShapeDtypeStruct((NC, NROWS, D), jnp.float32),
      mesh=mesh,
      compiler_params=pltpu.CompilerParams(use_tc_tiling_on_sc=False),
      scratch_types=[
          pltpu.VMEM((n0, ch), jnp.int32),
          pltpu.VMEM((n0, ch), jnp.int32),
          pltpu.VMEM((ch, D), jnp.float32),
          pltpu.VMEM((ch, D), jnp.float32),
          pltpu.VMEM_SHARED((NROWS, D), jnp.float32),
          pltpu.SemaphoreType.DMA,
          pltpu.SemaphoreType.DMA,
      ],
  )
  def agg(u_hbm, src_hbm, dst_hbm, zero_hbm, out_hbm,
          src_v, dst_v, msg0, msg1, acc, sem0, sem1):
    c = lax.axis_index("c")
    s = lax.axis_index("s")
    base = s * STRIPE
    start = jnp.where(c == 0, s * n0, NS * n0 + s * n1)

    pltpu.sync_copy(src_hbm.at[pl.ds(start, n0)], src_v)
    pltpu.sync_copy(dst_hbm.at[pl.ds(start, n0)], dst_v)
    pltpu.async_copy(u_hbm.at[src_v.at[0]], msg0, sem0)
    pltpu.sync_copy(zero_hbm, msg1)
    for k in range(STRIPE // ch):
      pltpu.sync_copy(msg1, acc.at[pl.ds(base + k * ch, ch)])
    plsc.subcore_barrier()

    def pipeline(npair):
      def body(i, carry):
        a = 2 * i
        pltpu.make_async_copy(u_hbm.at[src_v.at[a]], msg0, sem0).wait()
        pltpu.async_copy(u_hbm.at[src_v.at[a + 1]], msg1, sem1)
        pltpu.sync_copy(msg0, acc.at[dst_v.at[a]], add=True)
        pltpu.make_async_copy(u_hbm.at[src_v.at[a + 1]], msg1, sem1).wait()

        @pl.when(i < npair - 1)
        def _():
          pltpu.async_copy(u_hbm.at[src_v.at[a + 2]], msg0, sem0)

        pltpu.sync_copy(msg1, acc.at[dst_v.at[a + 1]], add=True)
        return carry

      lax.fori_loop(0, npair, body, 0)

    @pl.when(c == 0)
    def _():
      pipeline(n0 // 2)

    @pl.when(c == 1)
    def _():
      pipeline(n1 // 2)

    plsc.subcore_barrier()

    for k in range(STRIPE // ch):
      pltpu.sync_copy(acc.at[pl.ds(base + k * ch, ch)], msg0)
      pltpu.sync_copy(msg0, out_hbm.at[c, pl.ds(base + k * ch, ch)])

  return agg


_agg_cache = {}


def _agg(u, src_flat, dst_flat, D):
  if D not in _agg_cache:
    _agg_cache[D] = _make_agg(D)
  plan = _PLAN[D]
  ch, n0, n1 = plan["ch"], plan["n0"], plan["n1"]
  nchunk = NS * (n0 + n1)
  ne = nchunk * ch
  pads = jnp.zeros(((nchunk + n0) * ch - N_EDGES,), jnp.int32)
  padd = jnp.full(((nchunk + n0) * ch - N_EDGES,), TRASH, jnp.int32)
  src2 = jnp.concatenate([src_flat, pads]).reshape(nchunk + n0, ch)
  dst2 = jnp.concatenate([dst_flat, padd]).reshape(nchunk + n0, ch)
  zero = jnp.zeros((ch, D), jnp.float32)
  del ne
  return _agg_cache[D](u, src2, dst2, zero)


R = 1000
GRID = (N_NODES // R,)


def _row_spec(d):
  return pl.BlockSpec((R, d), lambda i: (i, 0))


def _full_spec(shape):
  return pl.BlockSpec(shape, lambda i: tuple(0 for _ in shape))


def _tc_pre_body(d0_ref, d1_ref, x_ref, dinv_ref, u1_ref):
  deg = d0_ref[...] + d1_ref[...] + 1.0
  dv = lax.rsqrt(deg)
  dinv_ref[...] = dv
  u1_ref[...] = dv * x_ref[...]


def _tc_pre(d0, d1, x):
  return pl.pallas_call(
      _tc_pre_body,
      grid=GRID,
      in_specs=[_row_spec(1), _row_spec(1), _row_spec(128)],
      out_specs=[_row_spec(1), _row_spec(128)],
      out_shape=[
          jax.ShapeDtypeStruct((N_NODES, 1), jnp.float32),
          jax.ShapeDtypeStruct((N_NODES, 128), jnp.float32),
      ],
  )(d0, d1, x)


def _tc1_body(p0, p1, u, dinv, W1, b1, W2, o):
  dv = dinv[...]
  a = dv * (p0[...] + p1[...] + u[...])
  h = jnp.maximum(jnp.dot(a, W1[...], preferred_element_type=jnp.float32)
                  + b1[...], 0.0)
  o[...] = dv * jnp.dot(h, W2[...], preferred_element_type=jnp.float32)


def _tc1(p0, p1, u, dinv, W1, b1, W2):
  return pl.pallas_call(
      _tc1_body,
      grid=GRID,
      in_specs=[_row_spec(128), _row_spec(128), _row_spec(128), _row_spec(1),
                _full_spec((128, 256)), _full_spec((1, 256)),
                _full_spec((256, 128))],
      out_specs=_row_spec(128),
      out_shape=jax.ShapeDtypeStruct((N_NODES, 128), jnp.float32),
  )(p0, p1, u, dinv, W1, b1, W2)


def _tc_mid_body(p0, p1, u, dinv, b, Wn, o):
  dv = dinv[...]
  h = jnp.maximum(dv * (p0[...] + p1[...] + u[...]) + b[...], 0.0)
  o[...] = dv * jnp.dot(h, Wn[...], preferred_element_type=jnp.float32)


def _tc_mid(p0, p1, u, dinv, b, Wn):
  d = u.shape[1]
  dn = Wn.shape[1]
  return pl.pallas_call(
      _tc_mid_body,
      grid=GRID,
      in_specs=[_row_spec(d), _row_spec(d), _row_spec(d), _row_spec(1),
                _full_spec((1, d)), _full_spec((d, dn))],
      out_specs=_row_spec(dn),
      out_shape=jax.ShapeDtypeStruct((N_NODES, dn), jnp.float32),
  )(p0, p1, u, dinv, b, Wn)


def _tc_h4_body(p0, p1, u, dinv, b, o):
  dv = dinv[...]
  h = jnp.maximum(dv * (p0[...] + p1[...] + u[...]) + b[...], 0.0)
  o[...] = dv * h


def _tc_h4(p0, p1, u, dinv, b):
  d = u.shape[1]
  return pl.pallas_call(
      _tc_h4_body,
      grid=GRID,
      in_specs=[_row_spec(d), _row_spec(d), _row_spec(d), _row_spec(1),
                _full_spec((1, d))],
      out_specs=_row_spec(d),
      out_shape=jax.ShapeDtypeStruct((N_NODES, d), jnp.float32),
  )(p0, p1, u, dinv, b)


def _tc_fin_body(p0, p1, u, dinv, Wm, bm, Wl, bl, mu, ls):
  a = dinv[...] * (p0[...] + p1[...] + u[...])
  mu[...] = jnp.dot(a, Wm[...], preferred_element_type=jnp.float32) + bm[...]
  ls[...] = jnp.dot(a, Wl[...], preferred_element_type=jnp.float32) + bl[...]


def _tc_fin(p0, p1, u, dinv, Wm, bm, Wl, bl):
  return pl.pallas_call(
      _tc_fin_body,
      grid=GRID,
      in_specs=[_row_spec(32), _row_spec(32), _row_spec(32), _row_spec(1),
                _full_spec((32, 16)), _full_spec((1, 16)),
                _full_spec((32, 16)), _full_spec((1, 16))],
      out_specs=[_row_spec(16), _row_spec(16)],
      out_shape=[
          jax.ShapeDtypeStruct((N_NODES, 16), jnp.float32),
          jax.ShapeDtypeStruct((N_NODES, 16), jnp.float32),
      ],
  )(p0, p1, u, dinv, Wm, bm, Wl, bl)


def kernel(x, edge_index, W1, b1, W2, b2, W3, b3, W4, b4,
           W_mu, b_mu, W_logstd, b_logstd):
  src = edge_index[0].astype(jnp.int32)
  dst = edge_index[1].astype(jnp.int32)

  b1r = b1.reshape(1, -1)
  b2r = b2.reshape(1, -1)
  b3r = b3.reshape(1, -1)
  b4r = b4.reshape(1, -1)
  bmr = b_mu.reshape(1, -1)
  blr = b_logstd.reshape(1, -1)

  ones = jnp.ones((N_NODES, 16), jnp.float32)
  degp = _agg(ones, src, dst, 16)
  d0 = degp[0, :N_NODES, :1]
  d1 = degp[1, :N_NODES, :1]
  dinv, u1 = _tc_pre(d0, d1, x)

  p = _agg(u1, src, dst, 128)
  u2 = _tc1(p[0, :N_NODES], p[1, :N_NODES], u1, dinv, W1, b1r, W2)

  p = _agg(u2, src, dst, 128)
  u3 = _tc_mid(p[0, :N_NODES], p[1, :N_NODES], u2, dinv, b2r, W3)

  p = _agg(u3, src, dst, 64)
  u4 = _tc_mid(p[0, :N_NODES], p[1, :N_NODES], u3, dinv, b3r, W4)

  p = _agg(u4, src, dst, 32)
  u5 = _tc_h4(p[0, :N_NODES], p[1, :N_NODES], u4, dinv, b4r)

  p = _agg(u5, src, dst, 32)
  mu, logstd = _tc_fin(p[0, :N_NODES], p[1, :N_NODES], u5, dinv,
                       W_mu, bmr, W_logstd, blr)
  return (mu, logstd)
